# baseline (speedup 1.0000x reference)
import functools

import jax
import jax.numpy as jnp
from jax import lax
from jax.experimental import pallas as pl
from jax.experimental.pallas import tpu as pltpu
from jax.experimental.pallas import tpu_sc as plsc

HIDDEN = 1024
N_EXPERTS = 8
INTER = 1408
T = 4096

BT = 256
G = 2 * T // BT + N_EXPERTS
PAD = G * BT
NW = 32
A = 2 * T
APW = A // NW
SUB = 32
TPW = T // NW
CSUB = 16
CHUNK = 512


def _route_kernel(logits_ref, pos_ref, wrep_ref, meta_ref):
    logits = logits_ref[...]
    m = jnp.max(logits, axis=-1, keepdims=True)
    p = jnp.exp(logits - m)
    p = p / jnp.sum(p, axis=-1, keepdims=True)
    iota = lax.broadcasted_iota(jnp.int32, p.shape, 1)
    p1 = jnp.max(p, axis=-1, keepdims=True)
    is1 = p == p1
    first1 = iota == jnp.min(jnp.where(is1, iota, N_EXPERTS), axis=-1, keepdims=True)
    p_wo = jnp.where(first1, -jnp.inf, p)
    p2 = jnp.max(p_wo, axis=-1, keepdims=True)
    is2 = p_wo == p2
    first2 = iota == jnp.min(jnp.where(is2, iota, N_EXPERTS), axis=-1, keepdims=True)
    denom = p1 + p2
    S = first1.astype(jnp.float32) + first2.astype(jnp.float32)

    counts = jnp.sum(S, axis=0, keepdims=True)
    nb = jnp.floor((counts + (BT - 1.0)) * (1.0 / BT))
    ii = lax.broadcasted_iota(jnp.int32, (N_EXPERTS, N_EXPERTS), 0)
    jj = lax.broadcasted_iota(jnp.int32, (N_EXPERTS, N_EXPERTS), 1)
    bs = jnp.dot(nb, (ii < jj).astype(jnp.float32),
                 preferred_element_type=jnp.float32)
    base = bs * float(BT)

    biota = lax.broadcasted_iota(jnp.int32, (N_EXPERTS, 128), 1).astype(jnp.float32)
    be = jnp.sum((jnp.broadcast_to(bs.T, (N_EXPERTS, 128)) <= biota)
                 .astype(jnp.float32), axis=0) - 1.0
    total_blocks = jnp.sum(nb)
    lane = lax.broadcasted_iota(jnp.int32, (128,), 0)
    be = jnp.where(lane == 120, total_blocks, be)
    meta_ref[...] = be[None, :].astype(jnp.int32)

    wrep_ref[0:T, :] = jnp.broadcast_to(p1 / denom, (T, 16))
    wrep_ref[T:A, :] = jnp.broadcast_to(p2 / denom, (T, 16))

    tri = (lax.broadcasted_iota(jnp.int32, (CHUNK, CHUNK), 0)
           >= lax.broadcasted_iota(jnp.int32, (CHUNK, CHUNK), 1)).astype(jnp.float32)
    running = jnp.zeros((1, N_EXPERTS), jnp.float32)
    p0_chunks, p1_chunks = [], []
    for c in range(T // CHUNK):
        Sc = S[c * CHUNK:(c + 1) * CHUNK, :]
        rank = jnp.dot(tri, Sc, preferred_element_type=jnp.float32) + running
        running = running + jnp.sum(Sc, axis=0, keepdims=True)
        slot = jnp.broadcast_to(base, rank.shape) + rank - 1.0
        f1c = first1[c * CHUNK:(c + 1) * CHUNK, :]
        f2c = first2[c * CHUNK:(c + 1) * CHUNK, :]
        p0_chunks.append(jnp.sum(jnp.where(f1c, slot, 0.0), axis=1))
        p1_chunks.append(jnp.sum(jnp.where(f2c, slot, 0.0), axis=1))
    pos0 = jnp.concatenate(p0_chunks)
    pos1 = jnp.concatenate(p1_chunks)
    pos_ref[...] = jnp.stack([pos0, pos1]).astype(jnp.int32)


@functools.lru_cache(maxsize=1)
def _sc_kernels():
    mesh = plsc.VectorSubcoreMesh(core_axis_name="c", subcore_axis_name="s")

    ND = APW // SUB
    NC = TPW // CSUB

    @functools.partial(
        pl.kernel,
        out_type=jax.ShapeDtypeStruct((PAD, HIDDEN), jnp.float32),
        mesh=mesh,
        scratch_types=[
            pltpu.VMEM((2, SUB), jnp.int32),
            pltpu.VMEM((2, SUB, HIDDEN), jnp.float32),
            [pltpu.SemaphoreType.DMA] * 2,
            [pltpu.SemaphoreType.DMA] * 2,
        ],
    )
    def dispatch(x_hbm, pos_hbm, xg_hbm, idx_v, rows_v, rsems, wsems):
        wid = lax.axis_index("s") * 2 + lax.axis_index("c")
        base_a = wid * APW

        def start_reads(j, b):
            a0 = base_a + j * SUB
            t0 = lax.rem(a0, T)
            return [
                pltpu.async_copy(pos_hbm.at[pl.ds(a0, SUB)], idx_v.at[b], rsems[b]),
                pltpu.async_copy(x_hbm.at[pl.ds(t0, SUB)], rows_v.at[b], rsems[b]),
            ]

        reads = {0: start_reads(0, 0)}
        writes = {}
        for j in range(ND):
            b = j % 2
            if j + 1 < ND:
                if j - 1 >= 0:
                    for c in writes.pop(j - 1):
                        c.wait()
                reads[j + 1] = start_reads(j + 1, (j + 1) % 2)
            for c in reads.pop(j):
                c.wait()
            writes[j] = [
                pltpu.async_copy(rows_v.at[b], xg_hbm.at[idx_v.at[b]], wsems[b]),
            ]
        for j in sorted(writes):
            for c in writes[j]:
                c.wait()

    @functools.partial(
        pl.kernel,
        out_type=jax.ShapeDtypeStruct((T, HIDDEN), jnp.float32),
        mesh=mesh,
        scratch_types=[
            pltpu.VMEM((2, CSUB), jnp.int32),
            pltpu.VMEM((2, CSUB), jnp.int32),
            pltpu.VMEM((2, CSUB, 16), jnp.float32),
            pltpu.VMEM((2, CSUB, 16), jnp.float32),
            pltpu.VMEM((2, CSUB, HIDDEN), jnp.float32),
            pltpu.VMEM((2, CSUB, HIDDEN), jnp.float32),
            [pltpu.SemaphoreType.DMA] * 2,
            [pltpu.SemaphoreType.DMA] * 2,
        ],
    )
    def combine(yg_hbm, pos_hbm, wrep_hbm, out_hbm, idx0_v, idx1_v,
                w0_v, w1_v, r0_v, r1_v, gsems, osems):
        wid = lax.axis_index("s") * 2 + lax.axis_index("c")
        tw = wid * TPW

        def start_gathers(j, b):
            t0 = tw + j * CSUB
            pltpu.sync_copy(pos_hbm.at[pl.ds(t0, CSUB)], idx0_v.at[b])
            pltpu.sync_copy(pos_hbm.at[pl.ds(T + t0, CSUB)], idx1_v.at[b])
            return [
                pltpu.async_copy(yg_hbm.at[idx0_v.at[b]], r0_v.at[b], gsems[b]),
                pltpu.async_copy(yg_hbm.at[idx1_v.at[b]], r1_v.at[b], gsems[b]),
                pltpu.async_copy(wrep_hbm.at[pl.ds(t0, CSUB)], w0_v.at[b],
                                 gsems[b]),
                pltpu.async_copy(wrep_hbm.at[pl.ds(T + t0, CSUB)], w1_v.at[b],
                                 gsems[b]),
            ]

        gathers = {0: start_gathers(0, 0)}
        outs = {}
        for j in range(NC):
            b = j % 2
            if j + 1 < NC:
                if j - 1 >= 0:
                    outs.pop(j - 1).wait()
                gathers[j + 1] = start_gathers(j + 1, (j + 1) % 2)
            for c in gathers.pop(j):
                c.wait()

            def row_body(r, _):
                w0 = w0_v[b, r, pl.ds(0, 16)]
                w1 = w1_v[b, r, pl.ds(0, 16)]
                for q in range(4):
                    for u in range(16):
                        off = q * 256 + u * 16
                        r0_v[b, r, pl.ds(off, 16)] = (
                            w0 * r0_v[b, r, pl.ds(off, 16)]
                            + w1 * r1_v[b, r, pl.ds(off, 16)])
                return 0

            lax.fori_loop(0, CSUB, row_body, 0)
            outs[j] = pltpu.async_copy(
                r0_v.at[b], out_hbm.at[pl.ds(tw + j * CSUB, CSUB)], osems[b])
        for j in sorted(outs):
            outs[j].wait()

    return dispatch, combine


def _gmm_kernel(be_ref, nblk_ref, xg_ref, w1_ref, w2_ref, yg_ref):
    del be_ref

    @pl.when(pl.program_id(0) < nblk_ref[0])
    def _():
        h = jnp.dot(xg_ref[...], w1_ref[0].T, preferred_element_type=jnp.float32)
        h = h * jax.nn.sigmoid(h)
        yg_ref[...] = jnp.dot(h, w2_ref[0].T, preferred_element_type=jnp.float32)


@jax.jit
def kernel(x, router_logits, w1, w2):
    pos, wrep, meta = pl.pallas_call(
        _route_kernel,
        out_shape=[
            jax.ShapeDtypeStruct((2, T), jnp.int32),
            jax.ShapeDtypeStruct((A, 16), jnp.float32),
            jax.ShapeDtypeStruct((1, 128), jnp.int32),
        ],
    )(router_logits)
    be = meta[0, :G]
    nblk = meta[0, 120:121]
    pos_flat = pos.reshape(A)

    dispatch, combine = _sc_kernels()
    xg = dispatch(x, pos_flat)

    yg = pl.pallas_call(
        _gmm_kernel,
        grid_spec=pltpu.PrefetchScalarGridSpec(
            num_scalar_prefetch=2,
            grid=(G,),
            in_specs=[
                pl.BlockSpec((BT, HIDDEN), lambda g, be_r, nb_r: (g, 0)),
                pl.BlockSpec((1, INTER, HIDDEN),
                             lambda g, be_r, nb_r: (be_r[g], 0, 0)),
                pl.BlockSpec((1, HIDDEN, INTER),
                             lambda g, be_r, nb_r: (be_r[g], 0, 0)),
            ],
            out_specs=pl.BlockSpec((BT, HIDDEN), lambda g, be_r, nb_r: (g, 0)),
        ),
        out_shape=jax.ShapeDtypeStruct((PAD, HIDDEN), jnp.float32),
    )(be, nblk, xg, w1, w2)

    return combine(yg, pos_flat, wrep)


# speedup vs baseline: 1.0141x; 1.0141x over previous
import functools

import jax
import jax.numpy as jnp
from jax import lax
from jax.experimental import pallas as pl
from jax.experimental.pallas import tpu as pltpu
from jax.experimental.pallas import tpu_sc as plsc

HIDDEN = 1024
N_EXPERTS = 8
INTER = 1408
T = 4096

BT = 512
G = 2 * T // BT + N_EXPERTS
PAD = G * BT
NW = 32
A = 2 * T
APW = A // NW
SUB = 32
TPW = T // NW
CSUB = 16
CHUNK = 512


def _route_kernel(logits_ref, pos_ref, wrep_ref, meta_ref):
    logits = logits_ref[...]
    m = jnp.max(logits, axis=-1, keepdims=True)
    p = jnp.exp(logits - m)
    p = p / jnp.sum(p, axis=-1, keepdims=True)
    iota = lax.broadcasted_iota(jnp.int32, p.shape, 1)
    p1 = jnp.max(p, axis=-1, keepdims=True)
    is1 = p == p1
    first1 = iota == jnp.min(jnp.where(is1, iota, N_EXPERTS), axis=-1, keepdims=True)
    p_wo = jnp.where(first1, -jnp.inf, p)
    p2 = jnp.max(p_wo, axis=-1, keepdims=True)
    is2 = p_wo == p2
    first2 = iota == jnp.min(jnp.where(is2, iota, N_EXPERTS), axis=-1, keepdims=True)
    denom = p1 + p2
    S = first1.astype(jnp.float32) + first2.astype(jnp.float32)

    counts = jnp.sum(S, axis=0, keepdims=True)
    nb = jnp.floor((counts + (BT - 1.0)) * (1.0 / BT))
    ii = lax.broadcasted_iota(jnp.int32, (N_EXPERTS, N_EXPERTS), 0)
    jj = lax.broadcasted_iota(jnp.int32, (N_EXPERTS, N_EXPERTS), 1)
    bs = jnp.dot(nb, (ii < jj).astype(jnp.float32),
                 preferred_element_type=jnp.float32)
    base = bs * float(BT)

    biota = lax.broadcasted_iota(jnp.int32, (N_EXPERTS, 128), 1).astype(jnp.float32)
    be = jnp.sum((jnp.broadcast_to(bs.T, (N_EXPERTS, 128)) <= biota)
                 .astype(jnp.float32), axis=0) - 1.0
    total_blocks = jnp.sum(nb)
    lane = lax.broadcasted_iota(jnp.int32, (128,), 0)
    be = jnp.where(lane == 120, total_blocks, be)
    meta_ref[...] = be[None, :].astype(jnp.int32)

    wrep_ref[0:T, :] = jnp.broadcast_to(p1 / denom, (T, 16))
    wrep_ref[T:A, :] = jnp.broadcast_to(p2 / denom, (T, 16))

    tri = (lax.broadcasted_iota(jnp.int32, (CHUNK, CHUNK), 0)
           >= lax.broadcasted_iota(jnp.int32, (CHUNK, CHUNK), 1)).astype(jnp.float32)
    running = jnp.zeros((1, N_EXPERTS), jnp.float32)
    p0_chunks, p1_chunks = [], []
    for c in range(T // CHUNK):
        Sc = S[c * CHUNK:(c + 1) * CHUNK, :]
        rank = jnp.dot(tri, Sc, preferred_element_type=jnp.float32) + running
        running = running + jnp.sum(Sc, axis=0, keepdims=True)
        slot = jnp.broadcast_to(base, rank.shape) + rank - 1.0
        f1c = first1[c * CHUNK:(c + 1) * CHUNK, :]
        f2c = first2[c * CHUNK:(c + 1) * CHUNK, :]
        p0_chunks.append(jnp.sum(jnp.where(f1c, slot, 0.0), axis=1))
        p1_chunks.append(jnp.sum(jnp.where(f2c, slot, 0.0), axis=1))
    pos0 = jnp.concatenate(p0_chunks)
    pos1 = jnp.concatenate(p1_chunks)
    pos_ref[...] = jnp.stack([pos0, pos1]).astype(jnp.int32)


@functools.lru_cache(maxsize=1)
def _sc_kernels():
    mesh = plsc.VectorSubcoreMesh(core_axis_name="c", subcore_axis_name="s")

    ND = APW // SUB
    NC = TPW // CSUB

    @functools.partial(
        pl.kernel,
        out_type=jax.ShapeDtypeStruct((PAD, HIDDEN), jnp.float32),
        mesh=mesh,
        scratch_types=[
            pltpu.VMEM((2, SUB), jnp.int32),
            pltpu.VMEM((2, SUB, HIDDEN), jnp.float32),
            [pltpu.SemaphoreType.DMA] * 2,
            [pltpu.SemaphoreType.DMA] * 2,
        ],
    )
    def dispatch(x_hbm, pos_hbm, xg_hbm, idx_v, rows_v, rsems, wsems):
        wid = lax.axis_index("s") * 2 + lax.axis_index("c")
        base_a = wid * APW

        def start_reads(j, b):
            a0 = base_a + j * SUB
            t0 = lax.rem(a0, T)
            return [
                pltpu.async_copy(pos_hbm.at[pl.ds(a0, SUB)], idx_v.at[b], rsems[b]),
                pltpu.async_copy(x_hbm.at[pl.ds(t0, SUB)], rows_v.at[b], rsems[b]),
            ]

        reads = {0: start_reads(0, 0)}
        writes = {}
        for j in range(ND):
            b = j % 2
            if j + 1 < ND:
                if j - 1 >= 0:
                    for c in writes.pop(j - 1):
                        c.wait()
                reads[j + 1] = start_reads(j + 1, (j + 1) % 2)
            for c in reads.pop(j):
                c.wait()
            writes[j] = [
                pltpu.async_copy(rows_v.at[b], xg_hbm.at[idx_v.at[b]], wsems[b]),
            ]
        for j in sorted(writes):
            for c in writes[j]:
                c.wait()

    @functools.partial(
        pl.kernel,
        out_type=jax.ShapeDtypeStruct((T, HIDDEN), jnp.float32),
        mesh=mesh,
        scratch_types=[
            pltpu.VMEM((2, CSUB), jnp.int32),
            pltpu.VMEM((2, CSUB), jnp.int32),
            pltpu.VMEM((2, CSUB, 16), jnp.float32),
            pltpu.VMEM((2, CSUB, 16), jnp.float32),
            pltpu.VMEM((2, CSUB, HIDDEN), jnp.float32),
            pltpu.VMEM((2, CSUB, HIDDEN), jnp.float32),
            [pltpu.SemaphoreType.DMA] * 2,
            [pltpu.SemaphoreType.DMA] * 2,
        ],
    )
    def combine(yg_hbm, pos_hbm, wrep_hbm, out_hbm, idx0_v, idx1_v,
                w0_v, w1_v, r0_v, r1_v, gsems, osems):
        wid = lax.axis_index("s") * 2 + lax.axis_index("c")
        tw = wid * TPW

        def start_gathers(j, b):
            t0 = tw + j * CSUB
            pltpu.sync_copy(pos_hbm.at[pl.ds(t0, CSUB)], idx0_v.at[b])
            pltpu.sync_copy(pos_hbm.at[pl.ds(T + t0, CSUB)], idx1_v.at[b])
            return [
                pltpu.async_copy(yg_hbm.at[idx0_v.at[b]], r0_v.at[b], gsems[b]),
                pltpu.async_copy(yg_hbm.at[idx1_v.at[b]], r1_v.at[b], gsems[b]),
                pltpu.async_copy(wrep_hbm.at[pl.ds(t0, CSUB)], w0_v.at[b],
                                 gsems[b]),
                pltpu.async_copy(wrep_hbm.at[pl.ds(T + t0, CSUB)], w1_v.at[b],
                                 gsems[b]),
            ]

        gathers = {0: start_gathers(0, 0)}
        outs = {}
        for j in range(NC):
            b = j % 2
            if j + 1 < NC:
                if j - 1 >= 0:
                    outs.pop(j - 1).wait()
                gathers[j + 1] = start_gathers(j + 1, (j + 1) % 2)
            for c in gathers.pop(j):
                c.wait()

            def row_body(r, _):
                w0 = w0_v[b, r, pl.ds(0, 16)]
                w1 = w1_v[b, r, pl.ds(0, 16)]
                for q in range(4):
                    for u in range(16):
                        off = q * 256 + u * 16
                        r0_v[b, r, pl.ds(off, 16)] = (
                            w0 * r0_v[b, r, pl.ds(off, 16)]
                            + w1 * r1_v[b, r, pl.ds(off, 16)])
                return 0

            lax.fori_loop(0, CSUB, row_body, 0)
            outs[j] = pltpu.async_copy(
                r0_v.at[b], out_hbm.at[pl.ds(tw + j * CSUB, CSUB)], osems[b])
        for j in sorted(outs):
            outs[j].wait()

    return dispatch, combine


def _gmm_kernel(be_ref, nblk_ref, xg_ref, w1_ref, w2_ref, yg_ref):
    del be_ref

    @pl.when(pl.program_id(0) < nblk_ref[0])
    def _():
        h = jnp.dot(xg_ref[...], w1_ref[0].T, preferred_element_type=jnp.float32)
        h = h * jax.nn.sigmoid(h)
        yg_ref[...] = jnp.dot(h, w2_ref[0].T, preferred_element_type=jnp.float32)


@jax.jit
def kernel(x, router_logits, w1, w2):
    pos, wrep, meta = pl.pallas_call(
        _route_kernel,
        out_shape=[
            jax.ShapeDtypeStruct((2, T), jnp.int32),
            jax.ShapeDtypeStruct((A, 16), jnp.float32),
            jax.ShapeDtypeStruct((1, 128), jnp.int32),
        ],
    )(router_logits)
    be = meta[0, :G]
    nblk = meta[0, 120:121]
    pos_flat = pos.reshape(A)

    dispatch, combine = _sc_kernels()
    xg = dispatch(x, pos_flat)

    yg = pl.pallas_call(
        _gmm_kernel,
        grid_spec=pltpu.PrefetchScalarGridSpec(
            num_scalar_prefetch=2,
            grid=(G,),
            in_specs=[
                pl.BlockSpec((BT, HIDDEN), lambda g, be_r, nb_r: (g, 0)),
                pl.BlockSpec((1, INTER, HIDDEN),
                             lambda g, be_r, nb_r: (be_r[g], 0, 0)),
                pl.BlockSpec((1, HIDDEN, INTER),
                             lambda g, be_r, nb_r: (be_r[g], 0, 0)),
            ],
            out_specs=pl.BlockSpec((BT, HIDDEN), lambda g, be_r, nb_r: (g, 0)),
        ),
        out_shape=jax.ShapeDtypeStruct((PAD, HIDDEN), jnp.float32),
    )(be, nblk, xg, w1, w2)

    return combine(yg, pos_flat, wrep)


# speedup vs baseline: 1.0286x; 1.0144x over previous
import functools

import jax
import jax.numpy as jnp
from jax import lax
from jax.experimental import pallas as pl
from jax.experimental.pallas import tpu as pltpu
from jax.experimental.pallas import tpu_sc as plsc

HIDDEN = 1024
N_EXPERTS = 8
INTER = 1408
T = 4096

BT = 512
G = 2 * T // BT + N_EXPERTS
PAD = G * BT
NW = 32
A = 2 * T
APW = A // NW
SUB = 32
TPW = T // NW
CSUB = 16
CHUNK = 512


def _route_kernel(logits_ref, pos_ref, wrep_ref, meta_ref):
    logits = logits_ref[...]
    m = jnp.max(logits, axis=-1, keepdims=True)
    p = jnp.exp(logits - m)
    p = p / jnp.sum(p, axis=-1, keepdims=True)
    iota = lax.broadcasted_iota(jnp.int32, p.shape, 1)
    p1 = jnp.max(p, axis=-1, keepdims=True)
    is1 = p == p1
    first1 = iota == jnp.min(jnp.where(is1, iota, N_EXPERTS), axis=-1, keepdims=True)
    p_wo = jnp.where(first1, -jnp.inf, p)
    p2 = jnp.max(p_wo, axis=-1, keepdims=True)
    is2 = p_wo == p2
    first2 = iota == jnp.min(jnp.where(is2, iota, N_EXPERTS), axis=-1, keepdims=True)
    denom = p1 + p2
    S = first1.astype(jnp.float32) + first2.astype(jnp.float32)

    counts = jnp.sum(S, axis=0, keepdims=True)
    nb = jnp.floor((counts + (BT - 1.0)) * (1.0 / BT))
    ii = lax.broadcasted_iota(jnp.int32, (N_EXPERTS, N_EXPERTS), 0)
    jj = lax.broadcasted_iota(jnp.int32, (N_EXPERTS, N_EXPERTS), 1)
    bs = jnp.dot(nb, (ii < jj).astype(jnp.float32),
                 preferred_element_type=jnp.float32)
    base = bs * float(BT)

    biota = lax.broadcasted_iota(jnp.int32, (N_EXPERTS, 128), 1).astype(jnp.float32)
    be = jnp.sum((jnp.broadcast_to(bs.T, (N_EXPERTS, 128)) <= biota)
                 .astype(jnp.float32), axis=0) - 1.0
    total_blocks = jnp.sum(nb)
    lane = lax.broadcasted_iota(jnp.int32, (128,), 0)
    be = jnp.where(lane == 120, total_blocks, be)
    meta_ref[...] = be[None, :].astype(jnp.int32)

    wrep_ref[0:T, :] = jnp.broadcast_to(p1 / denom, (T, 16))
    wrep_ref[T:A, :] = jnp.broadcast_to(p2 / denom, (T, 16))

    tri = (lax.broadcasted_iota(jnp.int32, (CHUNK, CHUNK), 0)
           >= lax.broadcasted_iota(jnp.int32, (CHUNK, CHUNK), 1)).astype(jnp.float32)
    running = jnp.zeros((1, N_EXPERTS), jnp.float32)
    p0_chunks, p1_chunks = [], []
    for c in range(T // CHUNK):
        Sc = S[c * CHUNK:(c + 1) * CHUNK, :]
        rank = jnp.dot(tri, Sc, preferred_element_type=jnp.float32) + running
        running = running + jnp.sum(Sc, axis=0, keepdims=True)
        slot = jnp.broadcast_to(base, rank.shape) + rank - 1.0
        f1c = first1[c * CHUNK:(c + 1) * CHUNK, :]
        f2c = first2[c * CHUNK:(c + 1) * CHUNK, :]
        p0_chunks.append(jnp.sum(jnp.where(f1c, slot, 0.0), axis=1))
        p1_chunks.append(jnp.sum(jnp.where(f2c, slot, 0.0), axis=1))
    pos0 = jnp.concatenate(p0_chunks)
    pos1 = jnp.concatenate(p1_chunks)
    pos_ref[...] = jnp.stack([pos0, pos1]).astype(jnp.int32)


@functools.lru_cache(maxsize=1)
def _sc_kernels():
    mesh = plsc.VectorSubcoreMesh(core_axis_name="c", subcore_axis_name="s")

    ND = APW // SUB
    NC = TPW // CSUB

    @functools.partial(
        pl.kernel,
        out_type=jax.ShapeDtypeStruct((PAD, HIDDEN), jnp.float32),
        mesh=mesh,
        scratch_types=[
            pltpu.VMEM((2, SUB), jnp.int32),
            pltpu.VMEM((2, SUB, HIDDEN), jnp.float32),
            [pltpu.SemaphoreType.DMA] * 2,
            [pltpu.SemaphoreType.DMA] * 2,
        ],
    )
    def dispatch(x_hbm, pos_hbm, xg_hbm, idx_v, rows_v, rsems, wsems):
        wid = lax.axis_index("s") * 2 + lax.axis_index("c")
        base_a = wid * APW

        def start_reads(j, b):
            a0 = base_a + j * SUB
            k = lax.div(a0, T)
            t0 = lax.rem(a0, T)
            return [
                pltpu.async_copy(pos_hbm.at[k, pl.ds(t0, SUB)], idx_v.at[b],
                                 rsems[b]),
                pltpu.async_copy(x_hbm.at[pl.ds(t0, SUB)], rows_v.at[b], rsems[b]),
            ]

        reads = {0: start_reads(0, 0)}
        writes = {}
        for j in range(ND):
            b = j % 2
            if j + 1 < ND:
                if j - 1 >= 0:
                    for c in writes.pop(j - 1):
                        c.wait()
                reads[j + 1] = start_reads(j + 1, (j + 1) % 2)
            for c in reads.pop(j):
                c.wait()
            writes[j] = [
                pltpu.async_copy(rows_v.at[b], xg_hbm.at[idx_v.at[b]], wsems[b]),
            ]
        for j in sorted(writes):
            for c in writes[j]:
                c.wait()

    @functools.partial(
        pl.kernel,
        out_type=jax.ShapeDtypeStruct((T, HIDDEN), jnp.float32),
        mesh=mesh,
        scratch_types=[
            pltpu.VMEM((2, CSUB), jnp.int32),
            pltpu.VMEM((2, CSUB), jnp.int32),
            pltpu.VMEM((2, CSUB, 16), jnp.float32),
            pltpu.VMEM((2, CSUB, 16), jnp.float32),
            pltpu.VMEM((2, CSUB, HIDDEN), jnp.float32),
            pltpu.VMEM((2, CSUB, HIDDEN), jnp.float32),
            [pltpu.SemaphoreType.DMA] * 2,
            [pltpu.SemaphoreType.DMA] * 2,
        ],
    )
    def combine(yg_hbm, pos_hbm, wrep_hbm, out_hbm, idx0_v, idx1_v,
                w0_v, w1_v, r0_v, r1_v, gsems, osems):
        wid = lax.axis_index("s") * 2 + lax.axis_index("c")
        tw = wid * TPW

        def start_gathers(j, b):
            t0 = tw + j * CSUB
            pltpu.sync_copy(pos_hbm.at[0, pl.ds(t0, CSUB)], idx0_v.at[b])
            pltpu.sync_copy(pos_hbm.at[1, pl.ds(t0, CSUB)], idx1_v.at[b])
            return [
                pltpu.async_copy(yg_hbm.at[idx0_v.at[b]], r0_v.at[b], gsems[b]),
                pltpu.async_copy(yg_hbm.at[idx1_v.at[b]], r1_v.at[b], gsems[b]),
                pltpu.async_copy(wrep_hbm.at[pl.ds(t0, CSUB)], w0_v.at[b],
                                 gsems[b]),
                pltpu.async_copy(wrep_hbm.at[pl.ds(T + t0, CSUB)], w1_v.at[b],
                                 gsems[b]),
            ]

        gathers = {0: start_gathers(0, 0)}
        outs = {}
        for j in range(NC):
            b = j % 2
            if j + 1 < NC:
                if j - 1 >= 0:
                    outs.pop(j - 1).wait()
                gathers[j + 1] = start_gathers(j + 1, (j + 1) % 2)
            for c in gathers.pop(j):
                c.wait()

            def row_body(r, _):
                w0 = w0_v[b, r, pl.ds(0, 16)]
                w1 = w1_v[b, r, pl.ds(0, 16)]
                for q in range(4):
                    for u in range(16):
                        off = q * 256 + u * 16
                        r0_v[b, r, pl.ds(off, 16)] = (
                            w0 * r0_v[b, r, pl.ds(off, 16)]
                            + w1 * r1_v[b, r, pl.ds(off, 16)])
                return 0

            lax.fori_loop(0, CSUB, row_body, 0)
            outs[j] = pltpu.async_copy(
                r0_v.at[b], out_hbm.at[pl.ds(tw + j * CSUB, CSUB)], osems[b])
        for j in sorted(outs):
            outs[j].wait()

    return dispatch, combine


def _gmm_kernel(be_ref, nblk_ref, xg_ref, w1_ref, w2_ref, yg_ref):
    del be_ref

    @pl.when(pl.program_id(0) < nblk_ref[0, 120])
    def _():
        h = jnp.dot(xg_ref[...], w1_ref[0].T, preferred_element_type=jnp.float32)
        h = h * jax.nn.sigmoid(h)
        yg_ref[...] = jnp.dot(h, w2_ref[0].T, preferred_element_type=jnp.float32)


@jax.jit
def kernel(x, router_logits, w1, w2):
    pos, wrep, meta = pl.pallas_call(
        _route_kernel,
        out_shape=[
            jax.ShapeDtypeStruct((2, T), jnp.int32),
            jax.ShapeDtypeStruct((A, 16), jnp.float32),
            jax.ShapeDtypeStruct((1, 128), jnp.int32),
        ],
    )(router_logits)
    dispatch, combine = _sc_kernels()
    xg = dispatch(x, pos)

    yg = pl.pallas_call(
        _gmm_kernel,
        grid_spec=pltpu.PrefetchScalarGridSpec(
            num_scalar_prefetch=2,
            grid=(G,),
            in_specs=[
                pl.BlockSpec((BT, HIDDEN), lambda g, be_r, nb_r: (g, 0)),
                pl.BlockSpec((1, INTER, HIDDEN),
                             lambda g, be_r, nb_r: (be_r[0, g], 0, 0)),
                pl.BlockSpec((1, HIDDEN, INTER),
                             lambda g, be_r, nb_r: (be_r[0, g], 0, 0)),
            ],
            out_specs=pl.BlockSpec((BT, HIDDEN),
                                   lambda g, be_r, nb_r: (g, 0)),
        ),
        out_shape=jax.ShapeDtypeStruct((PAD, HIDDEN), jnp.float32),
    )(meta, meta, xg, w1, w2)

    return combine(yg, pos, wrep)
